# baseline (speedup 1.0000x reference)
import jax
import jax.numpy as jnp
from jax import lax
from jax.experimental import pallas as pl
from jax.experimental.pallas import tpu as pltpu
from jax.experimental.pallas import tpu_sc as plsc

SEQ = 8192
D_MODEL = 1024
NUM_CORES = 2
NUM_SUBCORES = 16
NUM_WORKERS = NUM_CORES * NUM_SUBCORES
ROWS_PER_WORKER = SEQ // NUM_WORKERS


def _copy_body(pe_hbm, out_hbm):
    wid = lax.axis_index("s") * NUM_CORES + lax.axis_index("c")
    base = wid * ROWS_PER_WORKER
    pltpu.sync_copy(
        pe_hbm.at[pl.ds(base, ROWS_PER_WORKER)],
        out_hbm.at[pl.ds(base, ROWS_PER_WORKER)],
    )


@jax.jit
def kernel(x, position_embeddings):
    mesh = plsc.VectorSubcoreMesh(core_axis_name="c", subcore_axis_name="s")
    out = pl.kernel(
        _copy_body,
        mesh=mesh,
        out_type=jax.ShapeDtypeStruct((SEQ, D_MODEL), jnp.float32),
    )(position_embeddings)
    return out[None]


# speedup vs baseline: 24.1103x; 24.1103x over previous
import jax
import jax.numpy as jnp
from jax import lax
from jax.experimental import pallas as pl
from jax.experimental.pallas import tpu as pltpu
from jax.experimental.pallas import tpu_sc as plsc

SEQ = 8192
D_MODEL = 1024
NUM_CORES = 2
NUM_SUBCORES = 16
NUM_WORKERS = NUM_CORES * NUM_SUBCORES
ROWS_PER_WORKER = SEQ // NUM_WORKERS
CHUNK_ROWS = 32
NUM_CHUNKS = ROWS_PER_WORKER // CHUNK_ROWS


def _copy_body(pe_hbm, out_hbm, buf0, buf1, isem0, isem1, osem0, osem1):
    wid = lax.axis_index("s") * NUM_CORES + lax.axis_index("c")
    base = wid * ROWS_PER_WORKER
    bufs = (buf0, buf1)
    isems = (isem0, isem1)
    osems = (osem0, osem1)

    in_h = {}
    out_h = {}
    in_h[0] = pltpu.async_copy(pe_hbm.at[pl.ds(base, CHUNK_ROWS)], bufs[0], isems[0])
    for j in range(NUM_CHUNKS):
        b = j % 2
        if j >= 1:
            out_h[j - 1].wait()
        if j + 1 < NUM_CHUNKS:
            nb = (j + 1) % 2
            in_h[j + 1] = pltpu.async_copy(
                pe_hbm.at[pl.ds(base + (j + 1) * CHUNK_ROWS, CHUNK_ROWS)],
                bufs[nb], isems[nb])
        in_h[j].wait()
        out_h[j] = pltpu.async_copy(
            bufs[b], out_hbm.at[pl.ds(base + j * CHUNK_ROWS, CHUNK_ROWS)], osems[b])
    out_h[NUM_CHUNKS - 1].wait()


@jax.jit
def kernel(x, position_embeddings):
    mesh = plsc.VectorSubcoreMesh(core_axis_name="c", subcore_axis_name="s")
    out = pl.kernel(
        _copy_body,
        mesh=mesh,
        out_type=jax.ShapeDtypeStruct((SEQ, D_MODEL), jnp.float32),
        scratch_types=[
            pltpu.VMEM((CHUNK_ROWS, D_MODEL), jnp.float32),
            pltpu.VMEM((CHUNK_ROWS, D_MODEL), jnp.float32),
            pltpu.SemaphoreType.DMA,
            pltpu.SemaphoreType.DMA,
            pltpu.SemaphoreType.DMA,
            pltpu.SemaphoreType.DMA,
        ],
    )(position_embeddings)
    return out[None]
